# baseline (speedup 1.0000x reference)
import functools

import jax
import jax.numpy as jnp
from jax import lax
from jax.experimental import pallas as pl
from jax.experimental.pallas import tpu as pltpu
from jax.experimental.pallas import tpu_sc as plsc

FEAT = 1024
CHUNK_ROWS = 128
NBUF = 3
DEPTH = 2

_info = plsc.get_sparse_core_info()
_NC = _info.num_cores


@functools.partial(jax.jit, static_argnames=("length",))
def _sc_copy(table, length):
    rows_per_c = length // _NC
    nch = rows_per_c // CHUNK_ROWS
    mesh = plsc.ScalarSubcoreMesh(axis_name="c", num_cores=_NC)

    scratch = [pltpu.VMEM_SHARED((NBUF, CHUNK_ROWS, FEAT), table.dtype)]
    scratch += [pltpu.SemaphoreType.DMA for _ in range(2 * NBUF)]

    @functools.partial(
        pl.kernel,
        mesh=mesh,
        out_type=jax.ShapeDtypeStruct((length, FEAT), table.dtype),
        scratch_types=scratch,
    )
    def body(table_hbm, out_hbm, bufs, *sems):
        in_sems, out_sems = sems[:NBUF], sems[NBUF:]
        cid = lax.axis_index("c")
        base = cid * rows_per_c

        def start_in(g):
            b = g % NBUF
            return pltpu.async_copy(
                table_hbm.at[pl.ds(base + g * CHUNK_ROWS, CHUNK_ROWS)],
                bufs.at[b],
                in_sems[b],
            )

        def start_out(g):
            b = g % NBUF
            return pltpu.async_copy(
                bufs.at[b],
                out_hbm.at[pl.ds(base + g * CHUNK_ROWS, CHUNK_ROWS)],
                out_sems[b],
            )

        in_h = {}
        out_h = {}
        out_waited = set()
        for g in range(min(DEPTH, nch)):
            in_h[g] = start_in(g)
        for g in range(nch):
            nxt = g + DEPTH
            if nxt < nch:
                prev = nxt - NBUF
                if prev >= 0:
                    out_h[prev].wait()
                    out_waited.add(prev)
                in_h[nxt] = start_in(nxt)
            in_h[g].wait()
            out_h[g] = start_out(g)
        for g in range(nch):
            if g not in out_waited:
                out_h[g].wait()

    return body(table)


def kernel(x, table):
    return _sc_copy(table, x.shape[1])


# speedup vs baseline: 1.3076x; 1.3076x over previous
import functools

import jax
import jax.numpy as jnp
from jax import lax
from jax.experimental import pallas as pl
from jax.experimental.pallas import tpu as pltpu
from jax.experimental.pallas import tpu_sc as plsc
from jax._src.pallas import mpmd

FEAT = 1024

_info = plsc.get_sparse_core_info()
_NC, _NS = _info.num_cores, _info.num_subcores
_NW = _NC * _NS

CR_T = 16
NB_T = 6
DP_T = 3

CR_S = 128
NB_S = 3
DP_S = 2

TEC_FRAC_NUM, TEC_FRAC_DEN = 9, 16


def _pipeline(src, dst, bufs, in_sems, out_sems, base, nch, cr, nbuf, depth):
    def start_in(g):
        b = g % nbuf
        return pltpu.async_copy(
            src.at[pl.ds(base + g * cr, cr)], bufs.at[b], in_sems[b]
        )

    def start_out(g):
        b = g % nbuf
        return pltpu.async_copy(
            bufs.at[b], dst.at[pl.ds(base + g * cr, cr)], out_sems[b]
        )

    in_h, out_h, out_waited = {}, {}, set()
    for g in range(min(depth, nch)):
        in_h[g] = start_in(g)
    for g in range(nch):
        nxt = g + depth
        if nxt < nch:
            prev = nxt - nbuf
            if prev >= 0:
                out_h[prev].wait()
                out_waited.add(prev)
            in_h[nxt] = start_in(nxt)
        in_h[g].wait()
        out_h[g] = start_out(g)
    for g in range(nch):
        if g not in out_waited:
            out_h[g].wait()


@functools.partial(jax.jit, static_argnames=("length",))
def _sc_copy(table, length):
    tec_rows = (length * TEC_FRAC_NUM // TEC_FRAC_DEN) // _NW * _NW
    scs_rows = length - tec_rows
    rows_per_w = tec_rows // _NW
    rows_per_c = scs_rows // _NC
    nch_t = rows_per_w // CR_T
    nch_s = rows_per_c // CR_S

    vec_mesh = plsc.VectorSubcoreMesh(core_axis_name="c", subcore_axis_name="s")
    scs_mesh = plsc.ScalarSubcoreMesh(axis_name="c", num_cores=_NC)

    def tec_fn(table_hbm, out_hbm, sp_bufs):
        del sp_bufs
        wid = lax.axis_index("s") * _NC + lax.axis_index("c")
        base = wid * rows_per_w

        def inner(bufs, *sems):
            _pipeline(
                table_hbm, out_hbm, bufs, sems[:NB_T], sems[NB_T:],
                base, nch_t, CR_T, NB_T, DP_T,
            )

        pl.run_scoped(
            inner,
            pltpu.VMEM((NB_T, CR_T, FEAT), table.dtype),
            *([pltpu.SemaphoreType.DMA] * (2 * NB_T)),
        )

    def scs_fn(table_hbm, out_hbm, sp_bufs):
        cid = lax.axis_index("c")
        base = tec_rows + cid * rows_per_c

        def inner(*sems):
            _pipeline(
                table_hbm, out_hbm, sp_bufs, sems[:NB_S], sems[NB_S:],
                base, nch_s, CR_S, NB_S, DP_S,
            )

        pl.run_scoped(
            inner,
            *([pltpu.SemaphoreType.DMA] * (2 * NB_S)),
        )

    run = mpmd.mpmd_map(
        [(scs_mesh, scs_fn), (vec_mesh, tec_fn)],
        out_types=jax.ShapeDtypeStruct((length, FEAT), table.dtype),
        scratch_types=[pltpu.VMEM_SHARED((NB_S, CR_S, FEAT), table.dtype)],
    )
    return run(table)


def kernel(x, table):
    return _sc_copy(table, x.shape[1])


# speedup vs baseline: 1.3243x; 1.0128x over previous
import functools

import jax
import jax.numpy as jnp
from jax import lax
from jax.experimental import pallas as pl
from jax.experimental.pallas import tpu as pltpu
from jax.experimental.pallas import tpu_sc as plsc

FEAT = 1024
CHUNK_ROWS = 32
NBUF = 3
DEPTH = 2

_info = plsc.get_sparse_core_info()
_NC, _NS = _info.num_cores, _info.num_subcores
_NW = _NC * _NS


@functools.partial(jax.jit, static_argnames=("length",))
def _sc_copy(table, length):
    rows_per_w = length // _NW
    nch = rows_per_w // CHUNK_ROWS
    mesh = plsc.VectorSubcoreMesh(core_axis_name="c", subcore_axis_name="s")

    scratch = [pltpu.VMEM((NBUF, CHUNK_ROWS, FEAT), table.dtype)]
    scratch += [pltpu.SemaphoreType.DMA for _ in range(2 * NBUF)]

    @functools.partial(
        pl.kernel,
        mesh=mesh,
        out_type=jax.ShapeDtypeStruct((length, FEAT), table.dtype),
        scratch_types=scratch,
    )
    def body(table_hbm, out_hbm, bufs, *sems):
        in_sems, out_sems = sems[:NBUF], sems[NBUF:]
        wid = lax.axis_index("s") * _NC + lax.axis_index("c")
        base = wid * rows_per_w

        def start_in(g):
            b = g % NBUF
            return pltpu.async_copy(
                table_hbm.at[pl.ds(base + g * CHUNK_ROWS, CHUNK_ROWS)],
                bufs.at[b],
                in_sems[b],
            )

        def start_out(g):
            b = g % NBUF
            return pltpu.async_copy(
                bufs.at[b],
                out_hbm.at[pl.ds(base + g * CHUNK_ROWS, CHUNK_ROWS)],
                out_sems[b],
            )

        in_h = {}
        out_h = {}
        out_waited = set()
        for g in range(min(DEPTH, nch)):
            in_h[g] = start_in(g)
        for g in range(nch):
            nxt = g + DEPTH
            if nxt < nch:
                prev = nxt - NBUF
                if prev >= 0:
                    out_h[prev].wait()
                    out_waited.add(prev)
                in_h[nxt] = start_in(nxt)
            in_h[g].wait()
            out_h[g] = start_out(g)
        for g in range(nch):
            if g not in out_waited:
                out_h[g].wait()

    return body(table)


def kernel(x, table):
    return _sc_copy(table, x.shape[1])
